# baseline (speedup 1.0000x reference)
import functools

import jax
import jax.numpy as jnp
import numpy as np
from jax import lax
from jax.experimental import pallas as pl
from jax.experimental.pallas import tpu as pltpu
from jax.experimental.pallas import tpu_sc as plsc

_THRESHOLD = 0.01
_HALF_W = 960.0
_HALF_H = 540.0
_NUM_CLIPS = 10
_CLIP_LEN = 100
_V = 17
_C = 3
_ROW = _V * _C

_NUM_SC = 2
_NUM_SUBCORES = 16
_NW = _NUM_SC * _NUM_SUBCORES
_VALID_ROWS = 2 * _NUM_CLIPS * _CLIP_LEN
_MAX_RPW = -(-_VALID_ROWS // _NW)
_BUF_ROWS = 64


def _rotl(x, r):
    return ((x << np.uint32(r)) | (x >> np.uint32(32 - r))).astype(np.uint32)


def _threefry2x32(k1, k2, x0, x1):
    rotations = ((13, 15, 26, 6), (17, 29, 16, 24))
    ks = (np.uint32(k1), np.uint32(k2),
          np.uint32(np.uint32(k1) ^ np.uint32(k2) ^ np.uint32(0x1BD11BDA)))
    x0 = (x0 + ks[0]).astype(np.uint32)
    x1 = (x1 + ks[1]).astype(np.uint32)
    for i in range(5):
        for r in rotations[i % 2]:
            x0 = (x0 + x1).astype(np.uint32)
            x1 = _rotl(x1, r) ^ x0
        x0 = (x0 + ks[(i + 1) % 3]).astype(np.uint32)
        x1 = (x1 + ks[(i + 2) % 3] + np.uint32(i + 1)).astype(np.uint32)
    return x0, x1


@functools.lru_cache(maxsize=None)
def _clip_starts(num_frames):
    c1 = np.zeros(2, np.uint32)
    c2 = np.arange(2, dtype=np.uint32)
    b1, b2 = _threefry2x32(np.uint32(0), np.uint32(1), c1, c2)
    assert 65536 % num_frames == 0
    c1 = np.zeros(_NUM_CLIPS, np.uint32)
    c2 = np.arange(_NUM_CLIPS, dtype=np.uint32)
    o1, o2 = _threefry2x32(b1[1], b2[1], c1, c2)
    return [int(v) for v in (o1 ^ o2) % np.uint32(num_frames)]


def _worker_segments(num_frames):
    starts = _clip_starts(num_frames)
    plans = []
    for w in range(_NW):
        lo = _VALID_ROWS * w // _NW
        hi = _VALID_ROWS * (w + 1) // _NW
        segs = []
        o = lo
        while o < hi:
            blk = o // _CLIP_LEN
            nc, m = blk // 2, blk % 2
            s0 = o % _CLIP_LEN
            n = min(hi, (blk + 1) * _CLIP_LEN) - o
            f0 = (starts[nc] + s0) % num_frames
            n1 = min(n, num_frames - f0)
            segs.append((m * num_frames + f0, o - lo, n1))
            if n1 < n:
                segs.append((m * num_frames, o - lo + n1, n - n1))
            o += n
        plans.append((lo, hi - lo, segs))
    return plans


def _sc_body(plans, table, out, buf):
    wid = lax.axis_index("s") * _NUM_SC + lax.axis_index("c")

    for w, (lo, nr, segs) in enumerate(plans):
        @pl.when(wid == w)
        def _(segs=segs):
            for src, dst, n in segs:
                pltpu.sync_copy(table.at[pl.ds(src, n)], buf.at[pl.ds(dst, n)])

    iota = lax.iota(jnp.int32, 16)
    zero = jnp.zeros((16,), jnp.float32)
    for t in range(-(-_MAX_RPW * _V // 16)):
        trip = iota + (t * 16)
        r = trip // _V
        c0 = (trip - r * _V) * _C
        x = plsc.load_gather(buf, [r, c0])
        y = plsc.load_gather(buf, [r, c0 + 1])
        cf = plsc.load_gather(buf, [r, c0 + 2])
        m = cf <= _THRESHOLD
        plsc.store_scatter(buf, [r, c0], jnp.where(m, zero, (x - _HALF_W) / _HALF_W))
        plsc.store_scatter(buf, [r, c0 + 1], jnp.where(m, zero, (y - _HALF_H) / _HALF_H))

    for w, (lo, nr, segs) in enumerate(plans):
        @pl.when(wid == w)
        def _(lo=lo, nr=nr):
            pltpu.sync_copy(buf.at[pl.ds(0, nr)], out.at[pl.ds(lo, nr)])


def kernel(keypoint):
    num_person, num_frames, v, c = keypoint.shape
    assert (num_person, v, c) == (2, _V, _C)
    table = keypoint.reshape(num_person * num_frames, _ROW)
    plans = _worker_segments(num_frames)

    mesh = plsc.VectorSubcoreMesh(
        core_axis_name="c",
        subcore_axis_name="s",
        num_cores=_NUM_SC,
        num_subcores=_NUM_SUBCORES,
    )
    run = functools.partial(
        pl.kernel,
        mesh=mesh,
        compiler_params=pltpu.CompilerParams(
            use_tc_tiling_on_sc=False, needs_layout_passes=False
        ),
        out_type=jax.ShapeDtypeStruct((_VALID_ROWS, _ROW), jnp.float32),
        scratch_types=[
            pltpu.VMEM((_BUF_ROWS, _ROW), jnp.float32),
        ],
    )(functools.partial(_sc_body, plans))
    out = run(table)
    return out.reshape(1, _NUM_CLIPS, num_person, _CLIP_LEN, _V, _C)


# speedup vs baseline: 14.3643x; 14.3643x over previous
import functools

import jax
import jax.numpy as jnp
import numpy as np
from jax import lax
from jax.experimental import pallas as pl
from jax.experimental.pallas import tpu as pltpu


_THRESHOLD = 0.01
_HALF_W = 960.0
_HALF_H = 540.0
_NUM_CLIPS = 10
_CLIP_LEN = 100
_V = 17
_C = 3

import functools as _ft


def _rotl(x, r):
    return ((x << np.uint32(r)) | (x >> np.uint32(32 - r))).astype(np.uint32)


def _threefry2x32(k1, k2, x0, x1):
    rotations = ((13, 15, 26, 6), (17, 29, 16, 24))
    ks = (np.uint32(k1), np.uint32(k2),
          np.uint32(np.uint32(k1) ^ np.uint32(k2) ^ np.uint32(0x1BD11BDA)))
    x0 = (x0 + ks[0]).astype(np.uint32)
    x1 = (x1 + ks[1]).astype(np.uint32)
    for i in range(5):
        for r in rotations[i % 2]:
            x0 = (x0 + x1).astype(np.uint32)
            x1 = _rotl(x1, r) ^ x0
        x0 = (x0 + ks[(i + 1) % 3]).astype(np.uint32)
        x1 = (x1 + ks[(i + 2) % 3] + np.uint32(i + 1)).astype(np.uint32)
    return x0, x1


@functools.lru_cache(maxsize=None)
def _clip_starts(num_frames):
    c1 = np.zeros(2, np.uint32)
    c2 = np.arange(2, dtype=np.uint32)
    b1, b2 = _threefry2x32(np.uint32(0), np.uint32(1), c1, c2)
    assert 65536 % num_frames == 0
    c1 = np.zeros(_NUM_CLIPS, np.uint32)
    c2 = np.arange(_NUM_CLIPS, dtype=np.uint32)
    o1, o2 = _threefry2x32(b1[1], b2[1], c1, c2)
    return [int(v) for v in (o1 ^ o2) % np.uint32(num_frames)]


def _body(starts, num_frames, inp, out):
    a = inp[...]
    for nc in range(_NUM_CLIPS):
        f0 = starts[nc]
        n1 = min(_CLIP_LEN, num_frames - f0)
        if n1 < _CLIP_LEN:
            blk = jnp.concatenate(
                [a[:, :, :, f0:f0 + n1], a[:, :, :, :_CLIP_LEN - n1]], axis=3)
        else:
            blk = a[:, :, :, f0:f0 + _CLIP_LEN]
        x = blk[:, 0]
        y = blk[:, 1]
        cf = blk[:, 2]
        m = cf <= _THRESHOLD
        out[nc, :, 0] = jnp.where(m, 0.0, (x - _HALF_W) / _HALF_W)
        out[nc, :, 1] = jnp.where(m, 0.0, (y - _HALF_H) / _HALF_H)
        out[nc, :, 2] = cf


def kernel(keypoint):
    num_person, num_frames, v, c = keypoint.shape
    assert (num_person, v, c) == (2, _V, _C)
    kt = jnp.transpose(keypoint, (2, 3, 0, 1))
    starts = _clip_starts(num_frames)

    out5 = pl.pallas_call(
        functools.partial(_body, starts, num_frames),
        out_shape=jax.ShapeDtypeStruct(
            (_NUM_CLIPS, _V, _C, num_person, _CLIP_LEN), jnp.float32),
    )(kt)
    return jnp.transpose(out5, (0, 3, 4, 1, 2))[None]


# speedup vs baseline: 14.4333x; 1.0048x over previous
import functools

import jax
import jax.numpy as jnp
import numpy as np
from jax.experimental import pallas as pl

_THRESHOLD = 0.01
_HALF_W = 960.0
_HALF_H = 540.0
_NUM_CLIPS = 10
_CLIP_LEN = 100
_V = 17
_C = 3


def _rotl(x, r):
    return ((x << np.uint32(r)) | (x >> np.uint32(32 - r))).astype(np.uint32)


def _threefry2x32(k1, k2, x0, x1):
    rotations = ((13, 15, 26, 6), (17, 29, 16, 24))
    ks = (np.uint32(k1), np.uint32(k2),
          np.uint32(np.uint32(k1) ^ np.uint32(k2) ^ np.uint32(0x1BD11BDA)))
    x0 = (x0 + ks[0]).astype(np.uint32)
    x1 = (x1 + ks[1]).astype(np.uint32)
    for i in range(5):
        for r in rotations[i % 2]:
            x0 = (x0 + x1).astype(np.uint32)
            x1 = _rotl(x1, r) ^ x0
        x0 = (x0 + ks[(i + 1) % 3]).astype(np.uint32)
        x1 = (x1 + ks[(i + 2) % 3] + np.uint32(i + 1)).astype(np.uint32)
    return x0, x1


@functools.lru_cache(maxsize=None)
def _clip_starts(num_frames):
    c1 = np.zeros(2, np.uint32)
    c2 = np.arange(2, dtype=np.uint32)
    b1, b2 = _threefry2x32(np.uint32(0), np.uint32(1), c1, c2)
    assert 65536 % num_frames == 0
    c1 = np.zeros(_NUM_CLIPS, np.uint32)
    c2 = np.arange(_NUM_CLIPS, dtype=np.uint32)
    o1, o2 = _threefry2x32(b1[1], b2[1], c1, c2)
    return [int(v) for v in (o1 ^ o2) % np.uint32(num_frames)]


def _body(starts, num_frames, inp, out):
    a = inp[...]
    for nc in range(_NUM_CLIPS):
        f0 = starts[nc]
        n1 = min(_CLIP_LEN, num_frames - f0)
        if n1 < _CLIP_LEN:
            blk = jnp.concatenate(
                [a[:, :, :, f0:f0 + n1], a[:, :, :, :_CLIP_LEN - n1]], axis=3)
        else:
            blk = a[:, :, :, f0:f0 + _CLIP_LEN]
        x = blk[:, 0]
        y = blk[:, 1]
        cf = blk[:, 2]
        m = cf <= _THRESHOLD
        out[nc, :, 0] = jnp.where(m, 0.0, (x - _HALF_W) / _HALF_W)
        out[nc, :, 1] = jnp.where(m, 0.0, (y - _HALF_H) / _HALF_H)
        out[nc, :, 2] = cf


def kernel(keypoint):
    num_person, num_frames, v, c = keypoint.shape
    assert (num_person, v, c) == (2, _V, _C)
    kt = jnp.transpose(keypoint, (2, 3, 0, 1))
    starts = _clip_starts(num_frames)

    out5 = pl.pallas_call(
        functools.partial(_body, starts, num_frames),
        out_shape=jax.ShapeDtypeStruct(
            (_NUM_CLIPS, _V, _C, num_person, _CLIP_LEN), jnp.float32),
    )(kt)
    return jnp.transpose(out5, (0, 3, 4, 1, 2))[None]
